# Initial kernel scaffold; baseline (speedup 1.0000x reference)
import jax
import jax.numpy as jnp
from jax.experimental import pallas as pl


def kernel(inputs):
    raise NotImplementedError("write your pallas kernel here")


# speedup vs baseline: 7.3348x; 7.3348x over previous
import jax
import jax.numpy as jnp
from jax.experimental import pallas as pl

_B = 4
_N = 2048
_S = 1024


def _fps_body(x_ref, y_ref, z_ref, out_ref):
    X = x_ref[...]
    Y = y_ref[...]
    Z = z_ref[...]
    lane = jax.lax.broadcasted_iota(jnp.int32, (_B, _N), 1)

    def dist_from(px, py, pz):
        dx = X - px
        dy = Y - py
        dz = Z - pz
        s = dx * dx + dy * dy
        s = s + dz * dz
        return jnp.sqrt(jnp.maximum(s, 1e-12))

    ds0 = dist_from(x_ref[:, 0:1], y_ref[:, 0:1], z_ref[:, 0:1])

    def body(k, ds):
        m = jnp.max(ds, axis=1, keepdims=True)
        elig = ds == m
        idx = jnp.min(jnp.where(elig, lane, _N), axis=1, keepdims=True)
        onehot = lane == idx
        px = jnp.sum(jnp.where(onehot, X, 0.0), axis=1, keepdims=True)
        py = jnp.sum(jnp.where(onehot, Y, 0.0), axis=1, keepdims=True)
        pz = jnp.sum(jnp.where(onehot, Z, 0.0), axis=1, keepdims=True)
        pt = jnp.concatenate([px, py, pz], axis=1)
        out_ref[pl.ds(k, 1), :, :] = pt[None]
        nd = dist_from(px, py, pz)
        return jnp.minimum(ds, nd)

    jax.lax.fori_loop(0, _S, body, ds0)


def kernel(inputs):
    xs = inputs[:, :, 0]
    ys = inputs[:, :, 1]
    zs = inputs[:, :, 2]
    out = pl.pallas_call(
        _fps_body,
        out_shape=jax.ShapeDtypeStruct((_S, _B, 3), jnp.float32),
    )(xs, ys, zs)
    return jnp.transpose(out, (1, 0, 2))


# speedup vs baseline: 11.8166x; 1.6110x over previous
import jax
import jax.numpy as jnp
from jax.experimental import pallas as pl

_B = 4
_N = 2048
_S = 1024


def _fps_body(x_ref, y_ref, z_ref, ox_ref, oy_ref, oz_ref):
    X = x_ref[...]
    Y = y_ref[...]
    Z = z_ref[...]
    gidx = jax.lax.broadcasted_iota(jnp.int32, (_B, _N), 1)
    gs = jnp.bitwise_or(jnp.int32(0x20000000), jax.lax.shift_left(gidx, 18))
    c16 = jnp.int32(0xFFFF)
    big = jnp.float32(4.0)

    def make_keys(V):
        b = jax.lax.bitcast_convert_type(V, jnp.int32)
        hi = jax.lax.shift_right_logical(b, 16)
        lo = jnp.bitwise_and(b, c16)
        return (
            jax.lax.bitcast_convert_type(jnp.bitwise_or(gs, hi), jnp.float32),
            jax.lax.bitcast_convert_type(jnp.bitwise_or(gs, lo), jnp.float32),
        )

    kxh, kxl = make_keys(X)
    kyh, kyl = make_keys(Y)
    kzh, kzl = make_keys(Z)

    def dist_from(px, py, pz):
        dx = X - px
        dy = Y - py
        dz = Z - pz
        s = dx * dx + dy * dy
        s = s + dz * dz
        return jnp.sqrt(jnp.maximum(s, 1e-12))

    ds0 = dist_from(x_ref[:, 0:1], y_ref[:, 0:1], z_ref[:, 0:1])

    def body(k, ds):
        m = jnp.max(ds, axis=1, keepdims=True)
        elig = ds == m

        def ext(kh, kl):
            rh = jnp.min(jnp.where(elig, kh, big), axis=1, keepdims=True)
            rl = jnp.min(jnp.where(elig, kl, big), axis=1, keepdims=True)
            rhb = jax.lax.bitcast_convert_type(rh, jnp.int32)
            rlb = jax.lax.bitcast_convert_type(rl, jnp.int32)
            bits = jnp.bitwise_or(
                jax.lax.shift_left(jnp.bitwise_and(rhb, c16), 16),
                jnp.bitwise_and(rlb, c16),
            )
            return jax.lax.bitcast_convert_type(bits, jnp.float32)

        px = ext(kxh, kxl)
        py = ext(kyh, kyl)
        pz = ext(kzh, kzl)
        ox_ref[pl.ds(k, 1)] = px[None]
        oy_ref[pl.ds(k, 1)] = py[None]
        oz_ref[pl.ds(k, 1)] = pz[None]
        return jnp.minimum(ds, dist_from(px, py, pz))

    jax.lax.fori_loop(0, _S, body, ds0)


def kernel(inputs):
    xs = inputs[:, :, 0]
    ys = inputs[:, :, 1]
    zs = inputs[:, :, 2]
    shape = jax.ShapeDtypeStruct((_S, _B, 1), jnp.float32)
    ox, oy, oz = pl.pallas_call(
        _fps_body,
        out_shape=(shape, shape, shape),
    )(xs, ys, zs)
    out = jnp.concatenate([ox, oy, oz], axis=-1)
    return jnp.transpose(out, (1, 0, 2))
